# Initial kernel scaffold; baseline (speedup 1.0000x reference)
import jax
import jax.numpy as jnp
from jax.experimental import pallas as pl


def kernel(rgb_obj, stem_w, stem_b, head_w, head_b, fc_w, fc_b, proj_w, proj_b, fc_bn0_w1, fc_bn0_b1, fc_bn0_w2, fc_bn0_b2, fc_bn0_w3, fc_bn0_b3, fc_bn1_w1, fc_bn1_b1, fc_bn1_w2, fc_bn1_b2, fc_bn1_w3, fc_bn1_b3, proj_bn0_w1, proj_bn0_b1, proj_bn0_w2, proj_bn0_b2, proj_bn0_w3, proj_bn0_b3, proj_bn1_w1, proj_bn1_b1, proj_bn1_w2, proj_bn1_b2, proj_bn1_w3, proj_bn1_b3):
    raise NotImplementedError("write your pallas kernel here")


# speedup vs baseline: 1.0266x; 1.0266x over previous
import functools

import jax
import jax.numpy as jnp
from jax.experimental import pallas as pl
from jax.experimental.pallas import tpu as pltpu

_VMEM = pltpu.MemorySpace.VMEM
_VMEM_LIMIT = 100 * 1024 * 1024


def _pinned(shape):
    imap = lambda *_: (0,) * len(shape)
    try:
        return pl.BlockSpec(shape, imap, pipeline_mode=pl.Buffered(1))
    except TypeError:
        return pl.BlockSpec(shape, imap)


def _mm(a16, w, b):
    return jnp.dot(a16, w, preferred_element_type=jnp.float32) + b


def _bottleneck(x32, x16, w1, b1, w2, b2, w3, b3):
    h = jnp.maximum(_mm(x16, w1, b1), 0.0)
    h = jnp.maximum(_mm(h.astype(jnp.bfloat16), w2, b2), 0.0)
    y = _mm(h.astype(jnp.bfloat16), w3, b3)
    out = x32 + y
    return out, out.astype(jnp.bfloat16)


def _stem_local(x_ref, sw, sb, aw1, ab1, aw2, ab2, aw3, ab3,
                bw1, bb1, bw2, bb2, bw3, bb3, pw, pb,
                local_ref, pool_ref, *, imgs, tokens):
    feat = sw.shape[1]
    f = jnp.dot(x_ref[...], sw[...], preferred_element_type=jnp.float32) + sb[...]
    pool_ref[...] = (f.reshape(imgs, tokens // 8, 8, feat)
                      .sum(axis=1).reshape(imgs * 8, feat))
    x32, x16 = f, f.astype(jnp.bfloat16)
    x32, x16 = _bottleneck(x32, x16, aw1[...], ab1[...], aw2[...], ab2[...],
                           aw3[...], ab3[...])
    x32, x16 = _bottleneck(x32, x16, bw1[...], bb1[...], bw2[...], bb2[...],
                           bw3[...], bb3[...])
    o = jnp.dot(x16, pw[...], preferred_element_type=jnp.float32) + pb[...]
    local_ref[...] = o.reshape(local_ref.shape)


def _global_concat(pool_ref, local_ref, hw, hb, aw1, ab1, aw2, ab2, aw3, ab3,
                   bw1, bb1, bw2, bb2, bw3, bb3, fw, fb,
                   out_ref, *, inv_tokens):
    feat = pool_ref.shape[1]
    imgs = out_ref.shape[0]
    pooled = pool_ref[...].reshape(imgs, 8, feat).sum(axis=1) * inv_tokens
    g32 = _mm(pooled.astype(jnp.bfloat16), hw[...], hb[...])
    g16 = g32.astype(jnp.bfloat16)
    g32, g16 = _bottleneck(g32, g16, aw1[...], ab1[...], aw2[...], ab2[...],
                           aw3[...], ab3[...])
    g32, g16 = _bottleneck(g32, g16, bw1[...], bb1[...], bw2[...], bb2[...],
                           bw3[...], bb3[...])
    g = _mm(g16, fw[...], fb[...])
    out_ref[:, 0:1, :] = g[:, None, :]
    out_ref[:, 1:, :] = local_ref[...]


def kernel(rgb_obj, stem_w, stem_b, head_w, head_b, fc_w, fc_b, proj_w, proj_b,
           fc_bn0_w1, fc_bn0_b1, fc_bn0_w2, fc_bn0_b2, fc_bn0_w3, fc_bn0_b3,
           fc_bn1_w1, fc_bn1_b1, fc_bn1_w2, fc_bn1_b2, fc_bn1_w3, fc_bn1_b3,
           proj_bn0_w1, proj_bn0_b1, proj_bn0_w2, proj_bn0_b2, proj_bn0_w3,
           proj_bn0_b3, proj_bn1_w1, proj_bn1_b1, proj_bn1_w2, proj_bn1_b2,
           proj_bn1_w3, proj_bn1_b3):
    B, C, H, W = rgb_obj.shape
    win = 16
    ph, pw_n = H // win, W // win
    T = ph * pw_n
    K = C * win * win
    feat = stem_w.shape[1]
    latent = proj_w.shape[1]
    cmid = feat // 4

    x = rgb_obj.astype(jnp.bfloat16).reshape(B, C, ph, win, pw_n, win)
    x = x.transpose(0, 2, 4, 1, 3, 5).reshape(B * T, K)

    bf = jnp.bfloat16
    imgs1 = min(4, B)
    tm = imgs1 * T
    grid1 = (B // imgs1,)

    in_specs1 = [
        pl.BlockSpec((tm, K), lambda b: (b, 0)),
        _pinned((K, feat)), _pinned((1, feat)),
        _pinned((feat, cmid)), _pinned((1, cmid)),
        _pinned((cmid, cmid)), _pinned((1, cmid)),
        _pinned((cmid, feat)), _pinned((1, feat)),
        _pinned((feat, cmid)), _pinned((1, cmid)),
        _pinned((cmid, cmid)), _pinned((1, cmid)),
        _pinned((cmid, feat)), _pinned((1, feat)),
        _pinned((feat, latent)), _pinned((1, latent)),
    ]
    args1 = [x, stem_w.astype(bf), stem_b,
             proj_bn0_w1.astype(bf), proj_bn0_b1, proj_bn0_w2.astype(bf),
             proj_bn0_b2, proj_bn0_w3.astype(bf), proj_bn0_b3,
             proj_bn1_w1.astype(bf), proj_bn1_b1, proj_bn1_w2.astype(bf),
             proj_bn1_b2, proj_bn1_w3.astype(bf), proj_bn1_b3,
             proj_w.astype(bf), proj_b]

    bneck_macs = feat * cmid + cmid * cmid + cmid * feat
    n = B * T
    flops1 = 2 * n * (K * feat + 2 * bneck_macs + feat * latent)
    bytes1 = n * K * 2 + n * latent * 4 + B * 8 * feat * 4 + 8 * 1024 * 1024

    local, pool = pl.pallas_call(
        functools.partial(_stem_local, imgs=imgs1, tokens=T),
        grid=grid1,
        in_specs=in_specs1,
        out_specs=(pl.BlockSpec((imgs1, T, latent), lambda b: (b, 0, 0)),
                   pl.BlockSpec((imgs1 * 8, feat), lambda b: (b, 0))),
        out_shape=(jax.ShapeDtypeStruct((B, T, latent), jnp.float32),
                   jax.ShapeDtypeStruct((B * 8, feat), jnp.float32)),
        compiler_params=pltpu.CompilerParams(
            dimension_semantics=("parallel",),
            vmem_limit_bytes=_VMEM_LIMIT),
        cost_estimate=pl.CostEstimate(flops=int(flops1), transcendentals=0,
                                      bytes_accessed=int(bytes1)),
    )(*args1)

    imgs2 = min(8, B)
    grid2 = (B // imgs2,)
    hid = head_w.shape[1]
    hmid = hid // 4

    in_specs2 = [
        pl.BlockSpec((imgs2 * 8, feat), lambda b: (b, 0)),
        pl.BlockSpec((imgs2, T, latent), lambda b: (b, 0, 0)),
        _pinned((feat, hid)), _pinned((1, hid)),
        _pinned((hid, hmid)), _pinned((1, hmid)),
        _pinned((hmid, hmid)), _pinned((1, hmid)),
        _pinned((hmid, hid)), _pinned((1, hid)),
        _pinned((hid, hmid)), _pinned((1, hmid)),
        _pinned((hmid, hmid)), _pinned((1, hmid)),
        _pinned((hmid, hid)), _pinned((1, hid)),
        _pinned((hid, latent)), _pinned((1, latent)),
    ]
    args2 = [pool, local, head_w.astype(bf), head_b,
             fc_bn0_w1.astype(bf), fc_bn0_b1, fc_bn0_w2.astype(bf), fc_bn0_b2,
             fc_bn0_w3.astype(bf), fc_bn0_b3,
             fc_bn1_w1.astype(bf), fc_bn1_b1, fc_bn1_w2.astype(bf), fc_bn1_b2,
             fc_bn1_w3.astype(bf), fc_bn1_b3,
             fc_w.astype(bf), fc_b]

    gb_macs = feat * hid + 2 * (hid * hmid + hmid * hmid + hmid * hid) \
        + hid * latent
    flops2 = 2 * B * gb_macs
    bytes2 = B * 8 * feat * 4 + n * latent * 8 + 2 * gb_macs

    out = pl.pallas_call(
        functools.partial(_global_concat, inv_tokens=1.0 / T),
        grid=grid2,
        in_specs=in_specs2,
        out_specs=pl.BlockSpec((imgs2, 1 + T, latent), lambda b: (b, 0, 0)),
        out_shape=jax.ShapeDtypeStruct((B, 1 + T, latent), jnp.float32),
        compiler_params=pltpu.CompilerParams(
            dimension_semantics=("parallel",),
            vmem_limit_bytes=_VMEM_LIMIT),
        cost_estimate=pl.CostEstimate(flops=int(flops2), transcendentals=0,
                                      bytes_accessed=int(bytes2)),
    )(*args2)
    return out


# speedup vs baseline: 1.5028x; 1.4640x over previous
import functools

import jax
import jax.numpy as jnp
from jax.experimental import pallas as pl
from jax.experimental.pallas import tpu as pltpu

_VMEM = pltpu.MemorySpace.VMEM
_VMEM_LIMIT = 100 * 1024 * 1024


def _pinned(shape):
    imap = lambda *_: (0,) * len(shape)
    try:
        return pl.BlockSpec(shape, imap, pipeline_mode=pl.Buffered(1))
    except TypeError:
        return pl.BlockSpec(shape, imap)


def _mm(a16, w, b):
    return jnp.dot(a16, w, preferred_element_type=jnp.float32) + b


def _bottleneck(x32, x16, w1, b1, w2, b2, w3, b3):
    h = jnp.maximum(_mm(x16, w1, b1), 0.0)
    h = jnp.maximum(_mm(h.astype(jnp.bfloat16), w2, b2), 0.0)
    y = _mm(h.astype(jnp.bfloat16), w3, b3)
    out = x32 + y
    return out, out.astype(jnp.bfloat16)


def _stem_local(x_ref, sw, sb, aw1, ab1, aw2, ab2, aw3, ab3,
                bw1, bb1, bw2, bb2, bw3, bb3, pw, pb,
                local_ref, pool_ref, *, imgs, tokens, win, chans):
    feat = sw.shape[1]
    ph = x_ref.shape[2]
    pw_n = x_ref.shape[3] // (win * win)
    t = x_ref[...].astype(jnp.bfloat16).reshape(imgs, chans, ph, win, pw_n, win)
    t = t.transpose(0, 2, 4, 1, 3, 5)
    p = t.reshape(imgs * tokens, chans * win * win)

    f = jnp.dot(p, sw[...], preferred_element_type=jnp.float32) + sb[...]
    pool_ref[...] = (f.reshape(imgs, tokens // 8, 8, feat)
                      .sum(axis=1).reshape(imgs * 8, feat))
    x32, x16 = f, f.astype(jnp.bfloat16)
    x32, x16 = _bottleneck(x32, x16, aw1[...], ab1[...], aw2[...], ab2[...],
                           aw3[...], ab3[...])
    x32, x16 = _bottleneck(x32, x16, bw1[...], bb1[...], bw2[...], bb2[...],
                           bw3[...], bb3[...])
    o = jnp.dot(x16, pw[...], preferred_element_type=jnp.float32) + pb[...]
    local_ref[...] = o.reshape(local_ref.shape)


def _global_concat(pool_ref, local_ref, hw, hb, aw1, ab1, aw2, ab2, aw3, ab3,
                   bw1, bb1, bw2, bb2, bw3, bb3, fw, fb,
                   out_ref, *, inv_tokens):
    feat = pool_ref.shape[1]
    imgs = out_ref.shape[0]
    pooled = pool_ref[...].reshape(imgs, 8, feat).sum(axis=1) * inv_tokens
    g32 = _mm(pooled.astype(jnp.bfloat16), hw[...], hb[...])
    g16 = g32.astype(jnp.bfloat16)
    g32, g16 = _bottleneck(g32, g16, aw1[...], ab1[...], aw2[...], ab2[...],
                           aw3[...], ab3[...])
    g32, g16 = _bottleneck(g32, g16, bw1[...], bb1[...], bw2[...], bb2[...],
                           bw3[...], bb3[...])
    g = _mm(g16, fw[...], fb[...])
    out_ref[:, 0:1, :] = g[:, None, :]
    out_ref[:, 1:, :] = local_ref[...]


def kernel(rgb_obj, stem_w, stem_b, head_w, head_b, fc_w, fc_b, proj_w, proj_b,
           fc_bn0_w1, fc_bn0_b1, fc_bn0_w2, fc_bn0_b2, fc_bn0_w3, fc_bn0_b3,
           fc_bn1_w1, fc_bn1_b1, fc_bn1_w2, fc_bn1_b2, fc_bn1_w3, fc_bn1_b3,
           proj_bn0_w1, proj_bn0_b1, proj_bn0_w2, proj_bn0_b2, proj_bn0_w3,
           proj_bn0_b3, proj_bn1_w1, proj_bn1_b1, proj_bn1_w2, proj_bn1_b2,
           proj_bn1_w3, proj_bn1_b3):
    B, C, H, W = rgb_obj.shape
    win = 16
    ph, pw_n = H // win, W // win
    T = ph * pw_n
    K = C * win * win
    feat = stem_w.shape[1]
    latent = proj_w.shape[1]
    cmid = feat // 4

    bf = jnp.bfloat16
    imgs1 = min(4, B)
    steps1 = B // imgs1
    grid1 = (steps1,)

    in_specs1 = [
        pl.BlockSpec((imgs1, C, ph, win * pw_n * win),
                     lambda b: (b, 0, 0, 0)),
        _pinned((K, feat)), _pinned((1, feat)),
        _pinned((feat, cmid)), _pinned((1, cmid)),
        _pinned((cmid, cmid)), _pinned((1, cmid)),
        _pinned((cmid, feat)), _pinned((1, feat)),
        _pinned((feat, cmid)), _pinned((1, cmid)),
        _pinned((cmid, cmid)), _pinned((1, cmid)),
        _pinned((cmid, feat)), _pinned((1, feat)),
        _pinned((feat, latent)), _pinned((1, latent)),
    ]
    args1 = [rgb_obj.reshape(B, C, ph, win * pw_n * win),
             stem_w.astype(bf), stem_b,
             proj_bn0_w1.astype(bf), proj_bn0_b1, proj_bn0_w2.astype(bf),
             proj_bn0_b2, proj_bn0_w3.astype(bf), proj_bn0_b3,
             proj_bn1_w1.astype(bf), proj_bn1_b1, proj_bn1_w2.astype(bf),
             proj_bn1_b2, proj_bn1_w3.astype(bf), proj_bn1_b3,
             proj_w.astype(bf), proj_b]

    bneck_macs = feat * cmid + cmid * cmid + cmid * feat
    n = B * T
    flops1 = 2 * n * (K * feat + 2 * bneck_macs + feat * latent)
    bytes1 = n * K * 4 + n * latent * 4 + B * 8 * feat * 4 + 8 * 1024 * 1024

    local, pool = pl.pallas_call(
        functools.partial(_stem_local, imgs=imgs1, tokens=T, win=win, chans=C),
        grid=grid1,
        in_specs=in_specs1,
        out_specs=(pl.BlockSpec((imgs1, T, latent), lambda b: (b, 0, 0)),
                   pl.BlockSpec((imgs1 * 8, feat), lambda b: (b, 0))),
        out_shape=(jax.ShapeDtypeStruct((B, T, latent), jnp.float32),
                   jax.ShapeDtypeStruct((B * 8, feat), jnp.float32)),
        compiler_params=pltpu.CompilerParams(
            dimension_semantics=("parallel",),
            vmem_limit_bytes=_VMEM_LIMIT),
        cost_estimate=pl.CostEstimate(flops=int(flops1), transcendentals=0,
                                      bytes_accessed=int(bytes1)),
    )(*args1)

    imgs2 = min(8, B)
    steps2 = B // imgs2
    grid2 = (steps2,)
    hid = head_w.shape[1]
    hmid = hid // 4

    in_specs2 = [
        pl.BlockSpec((imgs2 * 8, feat), lambda b: (b, 0)),
        pl.BlockSpec((imgs2, T, latent), lambda b: (b, 0, 0)),
        _pinned((feat, hid)), _pinned((1, hid)),
        _pinned((hid, hmid)), _pinned((1, hmid)),
        _pinned((hmid, hmid)), _pinned((1, hmid)),
        _pinned((hmid, hid)), _pinned((1, hid)),
        _pinned((hid, hmid)), _pinned((1, hmid)),
        _pinned((hmid, hmid)), _pinned((1, hmid)),
        _pinned((hmid, hid)), _pinned((1, hid)),
        _pinned((hid, latent)), _pinned((1, latent)),
    ]
    args2 = [pool, local, head_w.astype(bf), head_b,
             fc_bn0_w1.astype(bf), fc_bn0_b1, fc_bn0_w2.astype(bf), fc_bn0_b2,
             fc_bn0_w3.astype(bf), fc_bn0_b3,
             fc_bn1_w1.astype(bf), fc_bn1_b1, fc_bn1_w2.astype(bf), fc_bn1_b2,
             fc_bn1_w3.astype(bf), fc_bn1_b3,
             fc_w.astype(bf), fc_b]

    gb_macs = feat * hid + 2 * (hid * hmid + hmid * hmid + hmid * hid) \
        + hid * latent
    flops2 = 2 * B * gb_macs
    bytes2 = B * 8 * feat * 4 + n * latent * 8 + 2 * gb_macs

    out = pl.pallas_call(
        functools.partial(_global_concat, inv_tokens=1.0 / T),
        grid=grid2,
        in_specs=in_specs2,
        out_specs=pl.BlockSpec((imgs2, 1 + T, latent), lambda b: (b, 0, 0)),
        out_shape=jax.ShapeDtypeStruct((B, 1 + T, latent), jnp.float32),
        compiler_params=pltpu.CompilerParams(
            dimension_semantics=("parallel",),
            vmem_limit_bytes=_VMEM_LIMIT),
        cost_estimate=pl.CostEstimate(flops=int(flops2), transcendentals=0,
                                      bytes_accessed=int(bytes2)),
    )(*args2)
    return out


# speedup vs baseline: 1.8026x; 1.1995x over previous
import functools

import jax
import jax.numpy as jnp
from jax.experimental import pallas as pl
from jax.experimental.pallas import tpu as pltpu

_VMEM = pltpu.MemorySpace.VMEM
_VMEM_LIMIT = 100 * 1024 * 1024


def _pinned(shape):
    imap = lambda *_: (0,) * len(shape)
    try:
        return pl.BlockSpec(shape, imap, pipeline_mode=pl.Buffered(1))
    except TypeError:
        return pl.BlockSpec(shape, imap)


def _mm(a16, w, b):
    return jnp.dot(a16, w, preferred_element_type=jnp.float32) + b


def _bottleneck(x32, x16, w1, b1, w2, b2, w3, b3):
    h = jnp.maximum(_mm(x16, w1, b1), 0.0)
    h = jnp.maximum(_mm(h.astype(jnp.bfloat16), w2, b2), 0.0)
    y = _mm(h.astype(jnp.bfloat16), w3, b3)
    out = x32 + y
    return out, out.astype(jnp.bfloat16)


def _stem_local(x_ref, sw, sb, aw1, ab1, aw2, ab2, aw3, ab3,
                bw1, bb1, bw2, bb2, bw3, bb3, pw, pb,
                local_ref, pool_ref, *, imgs, tokens, win, chans):
    feat = sw.shape[1]
    ph = x_ref.shape[2] // win
    pw_n = x_ref.shape[3] // win
    t = x_ref[...].astype(jnp.bfloat16).reshape(imgs, chans, ph, win, pw_n, win)
    t = t.transpose(0, 2, 4, 1, 3, 5)
    p = t.reshape(imgs * tokens, chans * win * win)

    f = jnp.dot(p, sw[...], preferred_element_type=jnp.float32) + sb[...]
    pool_ref[...] = (f.reshape(imgs, tokens // 8, 8, feat)
                      .sum(axis=1).reshape(imgs * 8, feat))
    x32, x16 = f, f.astype(jnp.bfloat16)
    x32, x16 = _bottleneck(x32, x16, aw1[...], ab1[...], aw2[...], ab2[...],
                           aw3[...], ab3[...])
    x32, x16 = _bottleneck(x32, x16, bw1[...], bb1[...], bw2[...], bb2[...],
                           bw3[...], bb3[...])
    o = jnp.dot(x16, pw[...], preferred_element_type=jnp.float32) + pb[...]
    local_ref[...] = o.reshape(local_ref.shape)


def _global_concat(pool_ref, local_ref, hw, hb, aw1, ab1, aw2, ab2, aw3, ab3,
                   bw1, bb1, bw2, bb2, bw3, bb3, fw, fb,
                   out_ref, g_ref, *, inv_tokens, imgs):
    feat = pool_ref.shape[1]
    batch = pool_ref.shape[0] // 8

    @pl.when(pl.program_id(0) == 0)
    def _():
        pooled = pool_ref[...].reshape(batch, 8, feat).sum(axis=1) * inv_tokens
        g32 = _mm(pooled.astype(jnp.bfloat16), hw[...], hb[...])
        g16 = g32.astype(jnp.bfloat16)
        g32, g16 = _bottleneck(g32, g16, aw1[...], ab1[...], aw2[...], ab2[...],
                               aw3[...], ab3[...])
        g32, g16 = _bottleneck(g32, g16, bw1[...], bb1[...], bw2[...], bb2[...],
                               bw3[...], bb3[...])
        g_ref[...] = _mm(g16, fw[...], fb[...])

    b = pl.program_id(0)
    off = pl.multiple_of(b * imgs, 8)
    out_ref[:, 0:1, :] = g_ref[pl.ds(off, imgs), :][:, None, :]
    out_ref[:, 1:, :] = local_ref[...]


def kernel(rgb_obj, stem_w, stem_b, head_w, head_b, fc_w, fc_b, proj_w, proj_b,
           fc_bn0_w1, fc_bn0_b1, fc_bn0_w2, fc_bn0_b2, fc_bn0_w3, fc_bn0_b3,
           fc_bn1_w1, fc_bn1_b1, fc_bn1_w2, fc_bn1_b2, fc_bn1_w3, fc_bn1_b3,
           proj_bn0_w1, proj_bn0_b1, proj_bn0_w2, proj_bn0_b2, proj_bn0_w3,
           proj_bn0_b3, proj_bn1_w1, proj_bn1_b1, proj_bn1_w2, proj_bn1_b2,
           proj_bn1_w3, proj_bn1_b3):
    B, C, H, W = rgb_obj.shape
    win = 16
    ph, pw_n = H // win, W // win
    T = ph * pw_n
    K = C * win * win
    feat = stem_w.shape[1]
    latent = proj_w.shape[1]
    cmid = feat // 4

    bf = jnp.bfloat16
    imgs1 = min(4, B)
    steps1 = B // imgs1
    grid1 = (steps1,)

    in_specs1 = [
        pl.BlockSpec((imgs1, C, H, W), lambda b: (b, 0, 0, 0)),
        _pinned((K, feat)), _pinned((1, feat)),
        _pinned((feat, cmid)), _pinned((1, cmid)),
        _pinned((cmid, cmid)), _pinned((1, cmid)),
        _pinned((cmid, feat)), _pinned((1, feat)),
        _pinned((feat, cmid)), _pinned((1, cmid)),
        _pinned((cmid, cmid)), _pinned((1, cmid)),
        _pinned((cmid, feat)), _pinned((1, feat)),
        _pinned((feat, latent)), _pinned((1, latent)),
    ]
    args1 = [rgb_obj, stem_w.astype(bf), stem_b,
             proj_bn0_w1.astype(bf), proj_bn0_b1, proj_bn0_w2.astype(bf),
             proj_bn0_b2, proj_bn0_w3.astype(bf), proj_bn0_b3,
             proj_bn1_w1.astype(bf), proj_bn1_b1, proj_bn1_w2.astype(bf),
             proj_bn1_b2, proj_bn1_w3.astype(bf), proj_bn1_b3,
             proj_w.astype(bf), proj_b]

    bneck_macs = feat * cmid + cmid * cmid + cmid * feat
    n = B * T
    flops1 = 2 * n * (K * feat + 2 * bneck_macs + feat * latent)
    bytes1 = n * K * 4 + n * latent * 4 + B * 8 * feat * 4 + 8 * 1024 * 1024

    local, pool = pl.pallas_call(
        functools.partial(_stem_local, imgs=imgs1, tokens=T, win=win, chans=C),
        grid=grid1,
        in_specs=in_specs1,
        out_specs=(pl.BlockSpec((imgs1, T, latent), lambda b: (b, 0, 0)),
                   pl.BlockSpec((imgs1 * 8, feat), lambda b: (b, 0))),
        out_shape=(jax.ShapeDtypeStruct((B, T, latent), jnp.float32),
                   jax.ShapeDtypeStruct((B * 8, feat), jnp.float32)),
        compiler_params=pltpu.CompilerParams(
            dimension_semantics=("parallel",),
            vmem_limit_bytes=_VMEM_LIMIT),
        cost_estimate=pl.CostEstimate(flops=int(flops1), transcendentals=0,
                                      bytes_accessed=int(bytes1)),
    )(*args1)

    imgs2 = min(8, B)
    steps2 = B // imgs2
    grid2 = (steps2,)
    hid = head_w.shape[1]
    hmid = hid // 4

    in_specs2 = [
        _pinned((B * 8, feat)),
        pl.BlockSpec((imgs2, T, latent), lambda b: (b, 0, 0)),
        _pinned((feat, hid)), _pinned((1, hid)),
        _pinned((hid, hmid)), _pinned((1, hmid)),
        _pinned((hmid, hmid)), _pinned((1, hmid)),
        _pinned((hmid, hid)), _pinned((1, hid)),
        _pinned((hid, hmid)), _pinned((1, hmid)),
        _pinned((hmid, hmid)), _pinned((1, hmid)),
        _pinned((hmid, hid)), _pinned((1, hid)),
        _pinned((hid, latent)), _pinned((1, latent)),
    ]
    args2 = [pool, local, head_w.astype(bf), head_b,
             fc_bn0_w1.astype(bf), fc_bn0_b1, fc_bn0_w2.astype(bf), fc_bn0_b2,
             fc_bn0_w3.astype(bf), fc_bn0_b3,
             fc_bn1_w1.astype(bf), fc_bn1_b1, fc_bn1_w2.astype(bf), fc_bn1_b2,
             fc_bn1_w3.astype(bf), fc_bn1_b3,
             fc_w.astype(bf), fc_b]

    gb_macs = feat * hid + 2 * (hid * hmid + hmid * hmid + hmid * hid) \
        + hid * latent
    flops2 = 2 * B * gb_macs
    bytes2 = B * 8 * feat * 4 + n * latent * 8 + 2 * gb_macs

    out = pl.pallas_call(
        functools.partial(_global_concat, inv_tokens=1.0 / T, imgs=imgs2),
        grid=grid2,
        in_specs=in_specs2,
        out_specs=pl.BlockSpec((imgs2, 1 + T, latent), lambda b: (b, 0, 0)),
        out_shape=jax.ShapeDtypeStruct((B, 1 + T, latent), jnp.float32),
        scratch_shapes=[pltpu.VMEM((B, latent), jnp.float32)],
        compiler_params=pltpu.CompilerParams(
            dimension_semantics=("arbitrary",),
            vmem_limit_bytes=_VMEM_LIMIT),
        cost_estimate=pl.CostEstimate(flops=int(flops2), transcendentals=0,
                                      bytes_accessed=int(bytes2)),
    )(*args2)
    return out


# speedup vs baseline: 1.8833x; 1.0448x over previous
import functools

import jax
import jax.numpy as jnp
from jax.experimental import pallas as pl
from jax.experimental.pallas import tpu as pltpu

_VMEM = pltpu.MemorySpace.VMEM
_VMEM_LIMIT = 100 * 1024 * 1024


def _pinned(shape):
    imap = lambda *_: (0,) * len(shape)
    try:
        return pl.BlockSpec(shape, imap, pipeline_mode=pl.Buffered(1))
    except TypeError:
        return pl.BlockSpec(shape, imap)


def _mm(a16, w, b):
    return jnp.dot(a16, w, preferred_element_type=jnp.float32) + b


def _bottleneck(x32, x16, w1, b1, w2, b2, w3, b3):
    h = jnp.maximum(_mm(x16, w1, b1), 0.0)
    h = jnp.maximum(_mm(h.astype(jnp.bfloat16), w2, b2), 0.0)
    y = _mm(h.astype(jnp.bfloat16), w3, b3)
    out = x32 + y
    return out, out.astype(jnp.bfloat16)


def _stem_local(x_ref, sw, sb, aw1, ab1, aw2, ab2, aw3, ab3,
                bw1, bb1, bw2, bb2, bw3, bb3, pw, pb,
                local_ref, pool_ref, *, imgs, tokens, win, chans):
    feat = sw.shape[1]
    ph = x_ref.shape[2] // win
    pw_n = x_ref.shape[3] // win
    for i in range(imgs):
        t = x_ref[i].astype(jnp.bfloat16).reshape(chans, ph, win, pw_n, win)
        t = t.transpose(1, 3, 0, 2, 4)
        p = t.reshape(tokens, chans * win * win)

        f = jnp.dot(p, sw[...], preferred_element_type=jnp.float32) + sb[...]
        pool_ref[pl.ds(i * 8, 8), :] = f.reshape(tokens // 8, 8, feat).sum(axis=0)
        x32, x16 = f, f.astype(jnp.bfloat16)
        x32, x16 = _bottleneck(x32, x16, aw1[...], ab1[...], aw2[...], ab2[...],
                               aw3[...], ab3[...])
        x32, x16 = _bottleneck(x32, x16, bw1[...], bb1[...], bw2[...], bb2[...],
                               bw3[...], bb3[...])
        o = jnp.dot(x16, pw[...], preferred_element_type=jnp.float32) + pb[...]
        local_ref[i] = o.reshape(local_ref.shape[1:])


def _global_concat(pool_ref, local_ref, hw, hb, aw1, ab1, aw2, ab2, aw3, ab3,
                   bw1, bb1, bw2, bb2, bw3, bb3, fw, fb,
                   out_ref, g_ref, *, inv_tokens, imgs):
    feat = pool_ref.shape[1]
    batch = pool_ref.shape[0] // 8

    @pl.when(pl.program_id(0) == 0)
    def _():
        pooled = pool_ref[...].reshape(batch, 8, feat).sum(axis=1) * inv_tokens
        g32 = _mm(pooled.astype(jnp.bfloat16), hw[...], hb[...])
        g16 = g32.astype(jnp.bfloat16)
        g32, g16 = _bottleneck(g32, g16, aw1[...], ab1[...], aw2[...], ab2[...],
                               aw3[...], ab3[...])
        g32, g16 = _bottleneck(g32, g16, bw1[...], bb1[...], bw2[...], bb2[...],
                               bw3[...], bb3[...])
        g_ref[...] = _mm(g16, fw[...], fb[...])

    b = pl.program_id(0)
    off = pl.multiple_of(b * imgs, 8)
    out_ref[:, 0:1, :] = g_ref[pl.ds(off, imgs), :][:, None, :]
    out_ref[:, 1:, :] = local_ref[...]


def kernel(rgb_obj, stem_w, stem_b, head_w, head_b, fc_w, fc_b, proj_w, proj_b,
           fc_bn0_w1, fc_bn0_b1, fc_bn0_w2, fc_bn0_b2, fc_bn0_w3, fc_bn0_b3,
           fc_bn1_w1, fc_bn1_b1, fc_bn1_w2, fc_bn1_b2, fc_bn1_w3, fc_bn1_b3,
           proj_bn0_w1, proj_bn0_b1, proj_bn0_w2, proj_bn0_b2, proj_bn0_w3,
           proj_bn0_b3, proj_bn1_w1, proj_bn1_b1, proj_bn1_w2, proj_bn1_b2,
           proj_bn1_w3, proj_bn1_b3):
    B, C, H, W = rgb_obj.shape
    win = 16
    ph, pw_n = H // win, W // win
    T = ph * pw_n
    K = C * win * win
    feat = stem_w.shape[1]
    latent = proj_w.shape[1]
    cmid = feat // 4

    bf = jnp.bfloat16
    imgs1 = min(4, B)
    steps1 = B // imgs1
    grid1 = (steps1,)

    in_specs1 = [
        pl.BlockSpec((imgs1, C, H, W), lambda b: (b, 0, 0, 0)),
        _pinned((K, feat)), _pinned((1, feat)),
        _pinned((feat, cmid)), _pinned((1, cmid)),
        _pinned((cmid, cmid)), _pinned((1, cmid)),
        _pinned((cmid, feat)), _pinned((1, feat)),
        _pinned((feat, cmid)), _pinned((1, cmid)),
        _pinned((cmid, cmid)), _pinned((1, cmid)),
        _pinned((cmid, feat)), _pinned((1, feat)),
        _pinned((feat, latent)), _pinned((1, latent)),
    ]
    args1 = [rgb_obj, stem_w.astype(bf), stem_b,
             proj_bn0_w1.astype(bf), proj_bn0_b1, proj_bn0_w2.astype(bf),
             proj_bn0_b2, proj_bn0_w3.astype(bf), proj_bn0_b3,
             proj_bn1_w1.astype(bf), proj_bn1_b1, proj_bn1_w2.astype(bf),
             proj_bn1_b2, proj_bn1_w3.astype(bf), proj_bn1_b3,
             proj_w.astype(bf), proj_b]

    bneck_macs = feat * cmid + cmid * cmid + cmid * feat
    n = B * T
    flops1 = 2 * n * (K * feat + 2 * bneck_macs + feat * latent)
    bytes1 = n * K * 4 + n * latent * 4 + B * 8 * feat * 4 + 8 * 1024 * 1024

    local, pool = pl.pallas_call(
        functools.partial(_stem_local, imgs=imgs1, tokens=T, win=win, chans=C),
        grid=grid1,
        in_specs=in_specs1,
        out_specs=(pl.BlockSpec((imgs1, T, latent), lambda b: (b, 0, 0)),
                   pl.BlockSpec((imgs1 * 8, feat), lambda b: (b, 0))),
        out_shape=(jax.ShapeDtypeStruct((B, T, latent), jnp.float32),
                   jax.ShapeDtypeStruct((B * 8, feat), jnp.float32)),
        compiler_params=pltpu.CompilerParams(
            dimension_semantics=("parallel",),
            vmem_limit_bytes=_VMEM_LIMIT),
        cost_estimate=pl.CostEstimate(flops=int(flops1), transcendentals=0,
                                      bytes_accessed=int(bytes1)),
    )(*args1)

    imgs2 = min(8, B)
    steps2 = B // imgs2
    grid2 = (steps2,)
    hid = head_w.shape[1]
    hmid = hid // 4

    in_specs2 = [
        _pinned((B * 8, feat)),
        pl.BlockSpec((imgs2, T, latent), lambda b: (b, 0, 0)),
        _pinned((feat, hid)), _pinned((1, hid)),
        _pinned((hid, hmid)), _pinned((1, hmid)),
        _pinned((hmid, hmid)), _pinned((1, hmid)),
        _pinned((hmid, hid)), _pinned((1, hid)),
        _pinned((hid, hmid)), _pinned((1, hmid)),
        _pinned((hmid, hmid)), _pinned((1, hmid)),
        _pinned((hmid, hid)), _pinned((1, hid)),
        _pinned((hid, latent)), _pinned((1, latent)),
    ]
    args2 = [pool, local, head_w.astype(bf), head_b,
             fc_bn0_w1.astype(bf), fc_bn0_b1, fc_bn0_w2.astype(bf), fc_bn0_b2,
             fc_bn0_w3.astype(bf), fc_bn0_b3,
             fc_bn1_w1.astype(bf), fc_bn1_b1, fc_bn1_w2.astype(bf), fc_bn1_b2,
             fc_bn1_w3.astype(bf), fc_bn1_b3,
             fc_w.astype(bf), fc_b]

    gb_macs = feat * hid + 2 * (hid * hmid + hmid * hmid + hmid * hid) \
        + hid * latent
    flops2 = 2 * B * gb_macs
    bytes2 = B * 8 * feat * 4 + n * latent * 8 + 2 * gb_macs

    out = pl.pallas_call(
        functools.partial(_global_concat, inv_tokens=1.0 / T, imgs=imgs2),
        grid=grid2,
        in_specs=in_specs2,
        out_specs=pl.BlockSpec((imgs2, 1 + T, latent), lambda b: (b, 0, 0)),
        out_shape=jax.ShapeDtypeStruct((B, 1 + T, latent), jnp.float32),
        scratch_shapes=[pltpu.VMEM((B, latent), jnp.float32)],
        compiler_params=pltpu.CompilerParams(
            dimension_semantics=("arbitrary",),
            vmem_limit_bytes=_VMEM_LIMIT),
        cost_estimate=pl.CostEstimate(flops=int(flops2), transcendentals=0,
                                      bytes_accessed=int(bytes2)),
    )(*args2)
    return out


# speedup vs baseline: 2.1559x; 1.1447x over previous
import functools

import jax
import jax.numpy as jnp
from jax.experimental import pallas as pl
from jax.experimental.pallas import tpu as pltpu

_VMEM = pltpu.MemorySpace.VMEM
_VMEM_LIMIT = 100 * 1024 * 1024


def _pinned(shape):
    imap = lambda *_: (0,) * len(shape)
    try:
        return pl.BlockSpec(shape, imap, pipeline_mode=pl.Buffered(1))
    except TypeError:
        return pl.BlockSpec(shape, imap)


def _mm(a16, w, b):
    return jnp.dot(a16, w, preferred_element_type=jnp.float32) + b


def _bottleneck(x32, x16, w1, b1, w2, b2, w3, b3):
    h = jnp.maximum(_mm(x16, w1, b1), 0.0)
    h = jnp.maximum(_mm(h.astype(jnp.bfloat16), w2, b2), 0.0)
    y = _mm(h.astype(jnp.bfloat16), w3, b3)
    out = x32 + y
    return out, out.astype(jnp.bfloat16)


def _stem_local(x_ref, sw, sb, aw1, ab1, aw2, ab2, aw3, ab3,
                bw1, bb1, bw2, bb2, bw3, bb3, pw, pb,
                local_ref, pool_ref,
                sw16, aw116, aw216, aw316, bw116, bw216, bw316, pw16,
                *, imgs, tokens, win, chans):
    feat = sw.shape[1]
    ph = x_ref.shape[2] // win
    pw_n = x_ref.shape[3] // win

    @pl.when(pl.program_id(0) == 0)
    def _():
        sw16[...] = sw[...].astype(jnp.bfloat16)
        aw116[...] = aw1[...].astype(jnp.bfloat16)
        aw216[...] = aw2[...].astype(jnp.bfloat16)
        aw316[...] = aw3[...].astype(jnp.bfloat16)
        bw116[...] = bw1[...].astype(jnp.bfloat16)
        bw216[...] = bw2[...].astype(jnp.bfloat16)
        bw316[...] = bw3[...].astype(jnp.bfloat16)
        pw16[...] = pw[...].astype(jnp.bfloat16)

    for i in range(imgs):
        t = x_ref[i].astype(jnp.bfloat16).reshape(chans, ph, win, pw_n, win)
        t = t.transpose(1, 3, 0, 2, 4)
        p = t.reshape(tokens, chans * win * win)

        f = jnp.dot(p, sw16[...], preferred_element_type=jnp.float32) + sb[...]
        pool_ref[pl.ds(i * 8, 8), :] = f.reshape(tokens // 8, 8, feat).sum(axis=0)
        x32, x16 = f, f.astype(jnp.bfloat16)
        x32, x16 = _bottleneck(x32, x16, aw116[...], ab1[...], aw216[...],
                               ab2[...], aw316[...], ab3[...])
        x32, x16 = _bottleneck(x32, x16, bw116[...], bb1[...], bw216[...],
                               bb2[...], bw316[...], bb3[...])
        o = jnp.dot(x16, pw16[...], preferred_element_type=jnp.float32) + pb[...]
        local_ref[i] = o.reshape(local_ref.shape[1:])


def _global_concat(pool_ref, local_ref, hw, hb, aw1, ab1, aw2, ab2, aw3, ab3,
                   bw1, bb1, bw2, bb2, bw3, bb3, fw, fb,
                   out_ref, g_ref, *, inv_tokens, imgs):
    feat = pool_ref.shape[1]
    batch = pool_ref.shape[0] // 8

    @pl.when(pl.program_id(0) == 0)
    def _():
        pooled = pool_ref[...].reshape(batch, 8, feat).sum(axis=1) * inv_tokens
        b16 = jnp.bfloat16
        g32 = _mm(pooled.astype(b16), hw[...].astype(b16), hb[...])
        g16 = g32.astype(b16)
        g32, g16 = _bottleneck(g32, g16, aw1[...].astype(b16), ab1[...],
                               aw2[...].astype(b16), ab2[...],
                               aw3[...].astype(b16), ab3[...])
        g32, g16 = _bottleneck(g32, g16, bw1[...].astype(b16), bb1[...],
                               bw2[...].astype(b16), bb2[...],
                               bw3[...].astype(b16), bb3[...])
        g_ref[...] = _mm(g16, fw[...].astype(b16), fb[...])

    b = pl.program_id(0)
    off = pl.multiple_of(b * imgs, 8)
    out_ref[:, 0:1, :] = g_ref[pl.ds(off, imgs), :][:, None, :]
    out_ref[:, 1:, :] = local_ref[...]


def kernel(rgb_obj, stem_w, stem_b, head_w, head_b, fc_w, fc_b, proj_w, proj_b,
           fc_bn0_w1, fc_bn0_b1, fc_bn0_w2, fc_bn0_b2, fc_bn0_w3, fc_bn0_b3,
           fc_bn1_w1, fc_bn1_b1, fc_bn1_w2, fc_bn1_b2, fc_bn1_w3, fc_bn1_b3,
           proj_bn0_w1, proj_bn0_b1, proj_bn0_w2, proj_bn0_b2, proj_bn0_w3,
           proj_bn0_b3, proj_bn1_w1, proj_bn1_b1, proj_bn1_w2, proj_bn1_b2,
           proj_bn1_w3, proj_bn1_b3):
    B, C, H, W = rgb_obj.shape
    win = 16
    ph, pw_n = H // win, W // win
    T = ph * pw_n
    K = C * win * win
    feat = stem_w.shape[1]
    latent = proj_w.shape[1]
    cmid = feat // 4

    bf = jnp.bfloat16
    imgs1 = min(4, B)
    steps1 = B // imgs1
    grid1 = (steps1,)

    in_specs1 = [
        pl.BlockSpec((imgs1, C, H, W), lambda b: (b, 0, 0, 0)),
        _pinned((K, feat)), _pinned((1, feat)),
        _pinned((feat, cmid)), _pinned((1, cmid)),
        _pinned((cmid, cmid)), _pinned((1, cmid)),
        _pinned((cmid, feat)), _pinned((1, feat)),
        _pinned((feat, cmid)), _pinned((1, cmid)),
        _pinned((cmid, cmid)), _pinned((1, cmid)),
        _pinned((cmid, feat)), _pinned((1, feat)),
        _pinned((feat, latent)), _pinned((1, latent)),
    ]
    args1 = [rgb_obj, stem_w, stem_b,
             proj_bn0_w1, proj_bn0_b1, proj_bn0_w2, proj_bn0_b2,
             proj_bn0_w3, proj_bn0_b3,
             proj_bn1_w1, proj_bn1_b1, proj_bn1_w2, proj_bn1_b2,
             proj_bn1_w3, proj_bn1_b3,
             proj_w, proj_b]

    bneck_macs = feat * cmid + cmid * cmid + cmid * feat
    n = B * T
    flops1 = 2 * n * (K * feat + 2 * bneck_macs + feat * latent)
    bytes1 = n * K * 4 + n * latent * 4 + B * 8 * feat * 4 + 8 * 1024 * 1024

    local, pool = pl.pallas_call(
        functools.partial(_stem_local, imgs=imgs1, tokens=T, win=win, chans=C),
        grid=grid1,
        in_specs=in_specs1,
        out_specs=(pl.BlockSpec((imgs1, T, latent), lambda b: (b, 0, 0)),
                   pl.BlockSpec((imgs1 * 8, feat), lambda b: (b, 0))),
        out_shape=(jax.ShapeDtypeStruct((B, T, latent), jnp.float32),
                   jax.ShapeDtypeStruct((B * 8, feat), jnp.float32)),
        scratch_shapes=[pltpu.VMEM((K, feat), bf),
                        pltpu.VMEM((feat, cmid), bf),
                        pltpu.VMEM((cmid, cmid), bf),
                        pltpu.VMEM((cmid, feat), bf),
                        pltpu.VMEM((feat, cmid), bf),
                        pltpu.VMEM((cmid, cmid), bf),
                        pltpu.VMEM((cmid, feat), bf),
                        pltpu.VMEM((feat, latent), bf)],
        compiler_params=pltpu.CompilerParams(
            dimension_semantics=("arbitrary",),
            vmem_limit_bytes=_VMEM_LIMIT),
        cost_estimate=pl.CostEstimate(flops=int(flops1), transcendentals=0,
                                      bytes_accessed=int(bytes1)),
    )(*args1)

    imgs2 = min(8, B)
    steps2 = B // imgs2
    grid2 = (steps2,)
    hid = head_w.shape[1]
    hmid = hid // 4

    in_specs2 = [
        _pinned((B * 8, feat)),
        pl.BlockSpec((imgs2, T, latent), lambda b: (b, 0, 0)),
        _pinned((feat, hid)), _pinned((1, hid)),
        _pinned((hid, hmid)), _pinned((1, hmid)),
        _pinned((hmid, hmid)), _pinned((1, hmid)),
        _pinned((hmid, hid)), _pinned((1, hid)),
        _pinned((hid, hmid)), _pinned((1, hmid)),
        _pinned((hmid, hmid)), _pinned((1, hmid)),
        _pinned((hmid, hid)), _pinned((1, hid)),
        _pinned((hid, latent)), _pinned((1, latent)),
    ]
    args2 = [pool, local, head_w, head_b,
             fc_bn0_w1, fc_bn0_b1, fc_bn0_w2, fc_bn0_b2, fc_bn0_w3, fc_bn0_b3,
             fc_bn1_w1, fc_bn1_b1, fc_bn1_w2, fc_bn1_b2, fc_bn1_w3, fc_bn1_b3,
             fc_w, fc_b]

    gb_macs = feat * hid + 2 * (hid * hmid + hmid * hmid + hmid * hid) \
        + hid * latent
    flops2 = 2 * B * gb_macs
    bytes2 = B * 8 * feat * 4 + n * latent * 8 + 2 * gb_macs

    out = pl.pallas_call(
        functools.partial(_global_concat, inv_tokens=1.0 / T, imgs=imgs2),
        grid=grid2,
        in_specs=in_specs2,
        out_specs=pl.BlockSpec((imgs2, 1 + T, latent), lambda b: (b, 0, 0)),
        out_shape=jax.ShapeDtypeStruct((B, 1 + T, latent), jnp.float32),
        scratch_shapes=[pltpu.VMEM((B, latent), jnp.float32)],
        compiler_params=pltpu.CompilerParams(
            dimension_semantics=("arbitrary",),
            vmem_limit_bytes=_VMEM_LIMIT),
        cost_estimate=pl.CostEstimate(flops=int(flops2), transcendentals=0,
                                      bytes_accessed=int(bytes2)),
    )(*args2)
    return out
